# Initial kernel scaffold; baseline (speedup 1.0000x reference)
import jax
import jax.numpy as jnp
from jax.experimental import pallas as pl


def kernel(h, edge_index):
    raise NotImplementedError("write your pallas kernel here")


# speedup vs baseline: 2.9600x; 2.9600x over previous
import functools

import jax
import jax.numpy as jnp
from jax import lax
from jax.experimental import pallas as pl
from jax.experimental.pallas import tpu as pltpu
from jax.experimental.pallas import tpu_sc as plsc

NW = 32
C = 80
D = 128
L = 16


def _make_sc_call(nchunk):
    mesh = plsc.VectorSubcoreMesh(
        core_axis_name="c", subcore_axis_name="s", num_cores=2,
        num_subcores=16)

    @functools.partial(
        pl.kernel,
        out_type=jax.ShapeDtypeStruct((NW, nchunk, C), jnp.float32),
        mesh=mesh,
        scratch_types=[
            pltpu.VMEM((nchunk, C), jnp.int32),
            pltpu.VMEM((nchunk, C), jnp.int32),
            pltpu.VMEM((C, D), jnp.float32),
            pltpu.VMEM((C, D), jnp.float32),
            pltpu.VMEM((nchunk, C), jnp.float32),
            pltpu.SemaphoreType.DMA,
            pltpu.SemaphoreType.DMA,
        ],
        compiler_params=pltpu.CompilerParams(needs_layout_passes=False),
    )
    def sc_dot(h_hbm, src_hbm, dst_hbm, out_hbm,
               sidx, didx, av, bv, ov, sema, semb):
        wid = lax.axis_index("s") * 2 + lax.axis_index("c")
        pltpu.sync_copy(src_hbm.at[wid], sidx)
        pltpu.sync_copy(dst_hbm.at[wid], didx)

        lane = lax.iota(jnp.int32, L)

        def chunk_body(c, carry):
            cpa = pltpu.async_copy(h_hbm.at[sidx.at[c]], av, sema)
            cpb = pltpu.async_copy(h_hbm.at[didx.at[c]], bv, semb)
            cpa.wait()
            cpb.wait()
            for g in range(C // L):
                res = jnp.zeros((L,), jnp.float32)
                for e in range(L):
                    row = g * L + e
                    acc = av[row, pl.ds(0, L)] * bv[row, pl.ds(0, L)]
                    for j in range(1, D // L):
                        acc = acc + (av[row, pl.ds(j * L, L)]
                                     * bv[row, pl.ds(j * L, L)])
                    res = jnp.where(lane == e, jnp.sum(acc), res)
                ov[c, pl.ds(g * L, L)] = res
            return carry

        lax.fori_loop(0, nchunk, chunk_body, 0)
        pltpu.sync_copy(ov, out_hbm.at[wid])

    return sc_dot


def kernel(h, edge_index):
    E = edge_index.shape[1]
    nchunk = E // (NW * C)
    src = edge_index[0].reshape(NW, nchunk, C)
    dst = edge_index[1].reshape(NW, nchunk, C)
    out = _make_sc_call(nchunk)(h, src, dst)
    return out.reshape(E)
